# Initial kernel scaffold; baseline (speedup 1.0000x reference)
import jax
import jax.numpy as jnp
from jax.experimental import pallas as pl


def kernel(x, w_gate, W1, b1, W2, b2):
    raise NotImplementedError("write your pallas kernel here")


# speedup vs baseline: 1.1651x; 1.1651x over previous
import jax
import jax.numpy as jnp
import numpy as np
from jax.experimental import pallas as pl
from jax.experimental.pallas import tpu as pltpu

D_MODEL = 1024
D_FF = 2048
E = 8
S = 2048
CAP = S // E
EPS = float(np.finfo(float).eps)
F32_INF_BITS = 0x7F800000


def _gating_body(x_ref, wg_ref, ru_ref, gates_ref, loss_ref):
    x = x_ref[...]
    logits = jnp.dot(x, wg_ref[...], preferred_element_type=jnp.float32)
    m = jnp.max(logits, axis=1, keepdims=True)
    ex = jnp.exp(logits - m)
    g = ex / jnp.sum(ex, axis=1, keepdims=True)

    cols = jax.lax.broadcasted_iota(jnp.int32, (S, E), 1)
    m1 = jnp.max(g, axis=1, keepdims=True)
    e1 = jnp.min(jnp.where(g == m1, cols, E), axis=1, keepdims=True)
    g_m = jnp.where(cols == e1, -jnp.inf, g)
    m2 = jnp.max(g_m, axis=1, keepdims=True)
    e2 = jnp.min(jnp.where(g_m == m2, cols, E), axis=1, keepdims=True)
    s12 = m1 + m2
    g1 = m1 / s12
    g2 = m2 / s12

    mean_g = jnp.mean(g, axis=0, keepdims=True)
    counts = jnp.sum((cols == e1).astype(jnp.float32), axis=0, keepdims=True)
    loss_ref[0, 0] = jnp.sum(counts / S * mean_g) / E * 0.1

    gcw1 = jnp.where(cols == e1, g1, 0.0)
    gcw2 = jnp.where(cols == e2, g2, 0.0)
    gcw2 = jnp.where(gcw2 > ru_ref[...], gcw2, 0.0)
    gcw = gcw1 + gcw2

    def bs_body(_, lohi):
        lo, hi = lohi
        mid = lo + (hi - lo) // 2
        t = jax.lax.bitcast_convert_type(mid, jnp.float32)
        cnt = jnp.sum((gcw > t).astype(jnp.int32), axis=0, keepdims=True)
        pred = cnt < CAP
        return jnp.where(pred, lo, mid + 1), jnp.where(pred, mid, hi)

    lo0 = jnp.zeros((1, E), jnp.int32)
    hi0 = jnp.full((1, E), F32_INF_BITS, jnp.int32)
    lo, _ = jax.lax.fori_loop(0, 31, bs_body, (lo0, hi0))
    t = jax.lax.bitcast_convert_type(lo, jnp.float32)

    n_gt = jnp.sum((gcw > t).astype(jnp.int32), axis=0, keepdims=True)
    quota = CAP - n_gt
    eq = (gcw == t) & (gcw > 0.0)
    eqf = eq.astype(jnp.float32)
    ri = jax.lax.broadcasted_iota(jnp.int32, (S, S), 0)
    ci = jax.lax.broadcasted_iota(jnp.int32, (S, S), 1)
    tri = (ci < ri).astype(jnp.float32)
    eq_prefix = jnp.dot(tri, eqf, preferred_element_type=jnp.float32)
    sel = (gcw > t) | (eq & (eq_prefix < quota.astype(jnp.float32)))
    gates_ref[...] = jnp.where(sel, gcw, 0.0)


SB = 1024


def _ffn_body(gates_ref, x_ref, w1_ref, b1_ref, w2_ref, b2_ref, y_ref):
    e = pl.program_id(1)
    cols = jax.lax.broadcasted_iota(jnp.int32, (SB, E), 1)
    gcol = jnp.sum(jnp.where(cols == e, gates_ref[...], 0.0), axis=1,
                   keepdims=True)
    x = x_ref[...]
    o = jnp.zeros((SB, D_MODEL), jnp.float32)
    nc = 4
    cf = D_FF // nc
    for c in range(nc):
        h = jnp.dot(x, w1_ref[0, :, pl.ds(c * cf, cf)],
                    preferred_element_type=jnp.float32)
        h = jnp.maximum(h + b1_ref[0, :, pl.ds(c * cf, cf)], 0.0)
        o = o + jnp.dot(h, w2_ref[0, pl.ds(c * cf, cf), :],
                        preferred_element_type=jnp.float32)
    contrib = gcol * (o + b2_ref[0])

    @pl.when(e == 0)
    def _():
        y_ref[...] = contrib

    @pl.when(e > 0)
    def _():
        y_ref[...] += contrib

    @pl.when(e == E - 1)
    def _():
        y = y_ref[...]
        y_ref[...] = jnp.where(y == 0.0, jnp.float32(EPS), y)


def kernel(x, w_gate, W1, b1, W2, b2):
    ru = jax.random.uniform(jax.random.key(42), (S, E), dtype=jnp.float32) / 2.0

    gates, loss = pl.pallas_call(
        _gating_body,
        out_shape=(
            jax.ShapeDtypeStruct((S, E), jnp.float32),
            jax.ShapeDtypeStruct((1, 1), jnp.float32),
        ),
        in_specs=[
            pl.BlockSpec((S, D_MODEL), lambda: (0, 0)),
            pl.BlockSpec((D_MODEL, E), lambda: (0, 0)),
            pl.BlockSpec((S, E), lambda: (0, 0)),
        ],
        out_specs=(
            pl.BlockSpec((S, E), lambda: (0, 0)),
            pl.BlockSpec((1, 1), lambda: (0, 0), memory_space=pltpu.SMEM),
        ),
    )(x, w_gate, ru)

    y = pl.pallas_call(
        _ffn_body,
        grid=(S // SB, E),
        out_shape=jax.ShapeDtypeStruct((S, D_MODEL), jnp.float32),
        in_specs=[
            pl.BlockSpec((SB, E), lambda sb, e: (sb, 0)),
            pl.BlockSpec((SB, D_MODEL), lambda sb, e: (sb, 0)),
            pl.BlockSpec((1, D_MODEL, D_FF), lambda sb, e: (e, 0, 0)),
            pl.BlockSpec((1, 1, D_FF), lambda sb, e: (e, 0, 0)),
            pl.BlockSpec((1, D_FF, D_MODEL), lambda sb, e: (e, 0, 0)),
            pl.BlockSpec((1, 1, D_MODEL), lambda sb, e: (e, 0, 0)),
        ],
        out_specs=pl.BlockSpec((SB, D_MODEL), lambda sb, e: (sb, 0)),
    )(gates, x, W1, b1.reshape(E, 1, D_FF), W2, b2.reshape(E, 1, D_MODEL))

    return y, loss.reshape(())
